# baseline (device time: 48633 ns/iter reference)
import os

import jax
import jax.numpy as jnp
from jax import lax
from jax.experimental import pallas as pl
from jax.experimental.pallas import tpu as pltpu

_VARIANT = os.environ.get("KERNEL_VARIANT", "full")

B, SQ, SKV, H, D = 8, 1, 512, 8, 64
NY = 4
BH = B * H
KH = SKV * H
SCALE = D ** -0.5


def kernel(Q, K, V):
    def body(q_ref, k_ref, v_ref, o_ref, comm_ref, send_sems, recv_sems):
        my_x = lax.axis_index("x")
        my_y = lax.axis_index("y")
        my_z = lax.axis_index("z")

        if _VARIANT == "loadonly":
            o_ref[...] = (k_ref[:, 0, :, :] + v_ref[:, 0, :, :]).reshape(
                B, SQ, H, D)
            return

        if _VARIANT == "full":
            barrier = pltpu.get_barrier_semaphore()
            for o in (1, 2, 3):
                pl.semaphore_signal(
                    barrier, inc=1,
                    device_id=(my_x, (my_y + o) % NY, my_z),
                    device_id_type=pl.DeviceIdType.MESH,
                )
            pl.semaphore_wait(barrier, 3)

        q2 = q_ref[...].reshape(BH, D)
        qt = jnp.transpose(q2) * SCALE
        eye = (lax.broadcasted_iota(jnp.int32, (H, H), 0)
               == lax.broadcasted_iota(jnp.int32, (H, H), 1)
               ).astype(jnp.float32)

        for b in range(B):
            kb = k_ref[b].reshape(KH, D)
            sb = jnp.dot(kb, qt[:, b * H:(b + 1) * H],
                         preferred_element_type=jnp.float32)
            s3 = sb.reshape(SKV, H, H)
            svec = jnp.sum(s3 * eye[None, :, :], axis=2)
            p = jnp.exp(svec)
            den = jnp.transpose(
                jnp.sum(p, axis=0, keepdims=True))
            num = jnp.sum(v_ref[b] * p[:, :, None], axis=0)
            comm_ref[0, b * H:(b + 1) * H, 0:D] = num
            comm_ref[0, b * H:(b + 1) * H, D:2 * D] = (
                jnp.broadcast_to(den, (H, D)))

        if _VARIANT == "compute":
            tot = comm_ref[0]
            o_ref[...] = (tot[:, 0:D] / tot[:, D:2 * D]).reshape(
                B, SQ, H, D)
            return

        rdmas = []
        for o in (1, 2, 3):
            rdma = pltpu.make_async_remote_copy(
                src_ref=comm_ref.at[0],
                dst_ref=comm_ref.at[o],
                send_sem=send_sems.at[o - 1],
                recv_sem=recv_sems.at[o - 1],
                device_id=(my_x, (my_y + o) % NY, my_z),
                device_id_type=pl.DeviceIdType.MESH,
            )
            rdma.start()
            rdmas.append(rdma)
        for rdma in rdmas:
            rdma.wait()

        tot = (comm_ref[0] + comm_ref[1] + comm_ref[2] + comm_ref[3])
        out = tot[:, 0:D] / tot[:, D:2 * D]
        o_ref[...] = out.reshape(B, SQ, H, D)

    return pl.pallas_call(
        body,
        out_shape=jax.ShapeDtypeStruct((B, SQ, H, D), jnp.float32),
        in_specs=[
            pl.BlockSpec(memory_space=pltpu.VMEM),
            pl.BlockSpec(memory_space=pltpu.VMEM),
            pl.BlockSpec(memory_space=pltpu.VMEM),
        ],
        out_specs=pl.BlockSpec(memory_space=pltpu.VMEM),
        scratch_shapes=[
            pltpu.VMEM((NY, BH, 2 * D), jnp.float32),
            pltpu.SemaphoreType.DMA((3,)),
            pltpu.SemaphoreType.DMA((3,)),
        ],
        compiler_params=pltpu.CompilerParams(
            collective_id=0 if _VARIANT == "full" else None,
            vmem_limit_bytes=100 * 1024 * 1024,
        ),
    )(Q, K, V)


# device time: 15279 ns/iter; 3.1830x vs baseline; 3.1830x over previous
import os

import jax
import jax.numpy as jnp
from jax import lax
from jax.experimental import pallas as pl
from jax.experimental.pallas import tpu as pltpu

_VARIANT = os.environ.get("KERNEL_VARIANT", "full")

B, SQ, SKV, H, D = 8, 1, 512, 8, 64
NY = 4
SCALE = D ** -0.5


def kernel(Q, K, V):
    def body(q_ref, k_ref, v_ref, o_ref, comm_ref, send_sems, recv_sems):
        my_x = lax.axis_index("x")
        my_y = lax.axis_index("y")
        my_z = lax.axis_index("z")

        if _VARIANT == "loadonly":
            o_ref[...] = (k_ref[:, :, :, 0] + v_ref[:, :, :, 0]).reshape(
                B, SQ, H, D)
            return

        if _VARIANT == "full":
            barrier = pltpu.get_barrier_semaphore()
            for o in (1, 2, 3):
                pl.semaphore_signal(
                    barrier, inc=1,
                    device_id=(my_x, (my_y + o) % NY, my_z),
                    device_id_type=pl.DeviceIdType.MESH,
                )
            pl.semaphore_wait(barrier, 3)

        q3 = q_ref[:, 0, :, :] * SCALE
        s = jnp.sum(k_ref[...] * q3[:, :, :, None], axis=2)
        p = jnp.exp(s)
        den = jnp.sum(p, axis=2)
        num = jnp.sum(v_ref[...] * p[:, :, None, :], axis=3)

        comm_ref[0, :, :, 0:D] = num
        comm_ref[0, :, :, D:2 * D] = jnp.broadcast_to(
            den[:, :, None], (B, H, D))

        if _VARIANT == "compute":
            tot = comm_ref[0]
            o_ref[...] = (tot[:, :, 0:D] / tot[:, :, D:2 * D]).reshape(
                B, SQ, H, D)
            return

        rdmas = []
        for o in (1, 2, 3):
            rdma = pltpu.make_async_remote_copy(
                src_ref=comm_ref.at[0],
                dst_ref=comm_ref.at[o],
                send_sem=send_sems.at[o - 1],
                recv_sem=recv_sems.at[o - 1],
                device_id=(my_x, (my_y + o) % NY, my_z),
                device_id_type=pl.DeviceIdType.MESH,
            )
            rdma.start()
            rdmas.append(rdma)
        for rdma in rdmas:
            rdma.wait()

        tot = (comm_ref[0] + comm_ref[1] + comm_ref[2] + comm_ref[3])
        out = tot[:, :, 0:D] / tot[:, :, D:2 * D]
        o_ref[...] = out.reshape(B, SQ, H, D)

    Kt = jnp.transpose(K, (0, 2, 3, 1))
    Vt = jnp.transpose(V, (0, 2, 3, 1))

    return pl.pallas_call(
        body,
        out_shape=jax.ShapeDtypeStruct((B, SQ, H, D), jnp.float32),
        in_specs=[
            pl.BlockSpec(memory_space=pltpu.VMEM),
            pl.BlockSpec(memory_space=pltpu.VMEM),
            pl.BlockSpec(memory_space=pltpu.VMEM),
        ],
        out_specs=pl.BlockSpec(memory_space=pltpu.VMEM),
        scratch_shapes=[
            pltpu.VMEM((NY, B, H, 2 * D), jnp.float32),
            pltpu.SemaphoreType.DMA((3,)),
            pltpu.SemaphoreType.DMA((3,)),
        ],
        compiler_params=pltpu.CompilerParams(
            collective_id=0 if _VARIANT == "full" else None,
            vmem_limit_bytes=100 * 1024 * 1024,
        ),
    )(Q, Kt, Vt)


# device time: 7095 ns/iter; 6.8545x vs baseline; 2.1535x over previous
import os

import jax
import jax.numpy as jnp
from jax import lax
from jax.experimental import pallas as pl
from jax.experimental.pallas import tpu as pltpu

_VARIANT = os.environ.get("KERNEL_VARIANT", "full")

B, SQ, SKV, H, D = 8, 1, 512, 8, 64
NY = 4
SCALE = D ** -0.5


def kernel(Q, K, V):
    def body(q_ref, k_ref, v_ref, o_ref, comm_ref, send_sems, recv_sems):
        my_x = lax.axis_index("x")
        my_y = lax.axis_index("y")
        my_z = lax.axis_index("z")

        if _VARIANT == "loadonly":
            o_ref[...] = (k_ref[:, :, :, 0] + v_ref[:, :, :, 0]).reshape(
                B, SQ, H, D)
            return

        if _VARIANT == "full":
            barrier = pltpu.get_barrier_semaphore()
            for o in (1, 2, 3):
                pl.semaphore_signal(
                    barrier, inc=1,
                    device_id=(my_x, (my_y + o) % NY, my_z),
                    device_id_type=pl.DeviceIdType.MESH,
                )

        q3 = q_ref[:, 0, :, :] * SCALE
        s = jnp.sum(k_ref[...] * q3[:, :, :, None], axis=2)
        p = jnp.exp(s)
        den = jnp.sum(p, axis=2)
        num = jnp.sum(v_ref[...] * p[:, :, None, :], axis=3)

        comm_ref[0, :, :, 0:D] = num
        comm_ref[0, :, :, D:D + 8] = jnp.broadcast_to(
            den[:, :, None], (B, H, 8))

        if _VARIANT == "compute":
            tot = comm_ref[0]
            o_ref[...] = (tot[:, :, 0:D] / tot[:, :, D:D + 1]).reshape(
                B, SQ, H, D)
            return

        pl.semaphore_wait(barrier, 3)

        rdmas = []
        for o in (1, 2, 3):
            rdma = pltpu.make_async_remote_copy(
                src_ref=comm_ref.at[0],
                dst_ref=comm_ref.at[o],
                send_sem=send_sems.at[o - 1],
                recv_sem=recv_sems.at[o - 1],
                device_id=(my_x, (my_y + o) % NY, my_z),
                device_id_type=pl.DeviceIdType.MESH,
            )
            rdma.start()
            rdmas.append(rdma)
        for rdma in rdmas:
            rdma.wait()

        tot = (comm_ref[0] + comm_ref[1] + comm_ref[2] + comm_ref[3])
        out = tot[:, :, 0:D] / tot[:, :, D:D + 1]
        o_ref[...] = out.reshape(B, SQ, H, D)

    Kt = jnp.transpose(K, (0, 2, 3, 1))
    Vt = jnp.transpose(V, (0, 2, 3, 1))

    return pl.pallas_call(
        body,
        out_shape=jax.ShapeDtypeStruct((B, SQ, H, D), jnp.float32),
        in_specs=[
            pl.BlockSpec(memory_space=pltpu.VMEM),
            pl.BlockSpec(memory_space=pltpu.VMEM),
            pl.BlockSpec(memory_space=pltpu.VMEM),
        ],
        out_specs=pl.BlockSpec(memory_space=pltpu.VMEM),
        scratch_shapes=[
            pltpu.VMEM((NY, B, H, D + 8), jnp.float32),
            pltpu.SemaphoreType.DMA((3,)),
            pltpu.SemaphoreType.DMA((3,)),
        ],
        compiler_params=pltpu.CompilerParams(
            collective_id=0 if _VARIANT == "full" else None,
            vmem_limit_bytes=100 * 1024 * 1024,
        ),
    )(Q, Kt, Vt)
